# baseline (speedup 1.0000x reference)
import jax
import jax.numpy as jnp
import numpy as np
from jax.experimental import pallas as pl
from jax.experimental.pallas import tpu as pltpu

_VMEM_LIMIT = 48 * 1024 * 1024


def _interp_mat(out_size, in_size):
    m = np.zeros((out_size, in_size), np.float32)
    for o in range(out_size):
        src = o * (in_size - 1) / (out_size - 1) if out_size > 1 else 0.0
        lo = int(np.floor(src))
        hi = min(lo + 1, in_size - 1)
        a = src - lo
        m[o, lo] += 1.0 - a
        m[o, hi] += a
    return jnp.asarray(m)


def _conv3x3_chunked(xp, w3, b, H, W):
    Cin = xp.shape[-1]
    K3 = 3 * Cin
    ch = 16 if H % 16 == 0 else H
    outs = []
    for r0 in range(0, H, ch):
        pw = jnp.concatenate(
            [xp[r0:r0 + ch + 2, kw:kw + W, :] for kw in range(3)], axis=-1)
        acc = jnp.dot(pw[0:ch].reshape(ch * W, K3), w3[0],
                      preferred_element_type=jnp.float32)
        acc = acc + jnp.dot(pw[1:ch + 1].reshape(ch * W, K3), w3[1],
                            preferred_element_type=jnp.float32)
        acc = acc + jnp.dot(pw[2:ch + 2].reshape(ch * W, K3), w3[2],
                            preferred_element_type=jnp.float32)
        outs.append(jnp.maximum(acc + b, 0.0))
    return jnp.concatenate(outs, axis=0)


def _pad_hw(x, H, W, C):
    zr = jnp.zeros((1, W, C), x.dtype)
    zc = jnp.zeros((H + 2, 1, C), x.dtype)
    xp = jnp.concatenate([zr, x, zr], axis=0)
    return jnp.concatenate([zc, xp, zc], axis=1)


def _up_block_kernel(x1_ref, x2_ref, wh_ref, wwt_ref,
                     w1_ref, s1_ref, b1_ref, w2_ref, s2_ref, b2_ref, o_ref):
    _, H1, W1, C1 = x1_ref.shape
    _, H2, W2, C2 = x2_ref.shape
    Cmid = w1_ref.shape[-1]
    Cout = w2_ref.shape[-1]
    bf16 = jnp.bfloat16

    w1f = (w1_ref[...] * s1_ref[0][None, None, None, :]).astype(bf16)
    w1f = w1f.reshape(3, 3 * (C1 + C2), Cmid)
    w2f = (w2_ref[...] * s2_ref[0][None, None, None, :]).astype(bf16)
    w2f = w2f.reshape(3, 3 * Cmid, Cout)
    b1 = b1_ref[...]
    b2 = b2_ref[...]

    x1hwc = x1_ref[0].reshape(H1, W1 * C1)
    t = jnp.dot(wh_ref[...], x1hwc,
                preferred_element_type=jnp.float32)
    t = jnp.transpose(t.reshape(H2, W1, C1), (0, 2, 1))
    u = jnp.dot(t.reshape(H2 * C1, W1).astype(bf16), wwt_ref[...],
                preferred_element_type=jnp.float32)
    up = jnp.transpose(u.reshape(H2, C1, W2), (0, 2, 1))

    xcat = jnp.concatenate([x2_ref[0], up.astype(bf16)], axis=-1)
    xp = _pad_hw(xcat, H2, W2, C1 + C2)

    mid = _conv3x3_chunked(xp, w1f, b1, H2, W2)
    mp = _pad_hw(mid.astype(bf16).reshape(H2, W2, Cmid), H2, W2, Cmid)
    y = _conv3x3_chunked(mp, w2f, b2, H2, W2)

    o_ref[0] = y.astype(bf16).reshape(H2, W2, Cout)


def kernel(x1_nchw, x2_nchw, w1, s1, b1, w2, s2, b2):
    N, C1, H1, W1 = x1_nchw.shape
    _, C2, H2, W2 = x2_nchw.shape
    Cin = C1 + C2
    Cmid = w1.shape[-1]
    Cout = w2.shape[-1]
    bf16 = jnp.bfloat16

    b1r = b1.reshape(1, Cmid).astype(jnp.float32)
    b2r = b2.reshape(1, Cout).astype(jnp.float32)
    s1r = s1.reshape(1, Cmid).astype(jnp.float32)
    s2r = s2.reshape(1, Cout).astype(jnp.float32)

    wh = _interp_mat(H2, H1).astype(bf16)
    wwt = _interp_mat(W2, W1).T.astype(bf16)

    x1h = jnp.transpose(x1_nchw, (0, 2, 3, 1)).astype(bf16)
    x2h = jnp.transpose(x2_nchw, (0, 2, 3, 1)).astype(bf16)

    yh = pl.pallas_call(
        _up_block_kernel,
        out_shape=jax.ShapeDtypeStruct((N, H2, W2, Cout), bf16),
        grid=(N,),
        in_specs=[
            pl.BlockSpec((1, H1, W1, C1), lambda n: (n, 0, 0, 0)),
            pl.BlockSpec((1, H2, W2, C2), lambda n: (n, 0, 0, 0)),
            pl.BlockSpec((H2, H1), lambda n: (0, 0)),
            pl.BlockSpec((W1, W2), lambda n: (0, 0)),
            pl.BlockSpec((3, 3, Cin, Cmid), lambda n: (0, 0, 0, 0)),
            pl.BlockSpec((1, Cmid), lambda n: (0, 0)),
            pl.BlockSpec((1, Cmid), lambda n: (0, 0)),
            pl.BlockSpec((3, 3, Cmid, Cout), lambda n: (0, 0, 0, 0)),
            pl.BlockSpec((1, Cout), lambda n: (0, 0)),
            pl.BlockSpec((1, Cout), lambda n: (0, 0)),
        ],
        out_specs=pl.BlockSpec((1, H2, W2, Cout), lambda n: (n, 0, 0, 0)),
        compiler_params=pltpu.CompilerParams(
            dimension_semantics=("parallel",),
            vmem_limit_bytes=_VMEM_LIMIT),
    )(x1h, x2h, wh, wwt, w1, s1r, b1r, w2, s2r, b2r)
    return jnp.transpose(yh, (0, 3, 1, 2)).astype(x2_nchw.dtype)


# speedup vs baseline: 1.1680x; 1.1680x over previous
import jax
import jax.numpy as jnp
import numpy as np
from jax.experimental import pallas as pl
from jax.experimental.pallas import tpu as pltpu

_VMEM_LIMIT = 48 * 1024 * 1024


def _interp_mat(out_size, in_size):
    m = np.zeros((out_size, in_size), np.float32)
    for o in range(out_size):
        src = o * (in_size - 1) / (out_size - 1) if out_size > 1 else 0.0
        lo = int(np.floor(src))
        hi = min(lo + 1, in_size - 1)
        a = src - lo
        m[o, lo] += 1.0 - a
        m[o, hi] += a
    return jnp.asarray(m)


def _conv3x3_chunked(xp, w3, b, H, W):
    Cin = xp.shape[-1]
    K3 = 3 * Cin
    ch = 16 if H % 16 == 0 else H
    outs = []
    for r0 in range(0, H, ch):
        pw = jnp.concatenate(
            [xp[r0:r0 + ch + 2, kw:kw + W, :] for kw in range(3)], axis=-1)
        acc = jnp.dot(pw[0:ch].reshape(ch * W, K3), w3[0],
                      preferred_element_type=jnp.float32)
        acc = acc + jnp.dot(pw[1:ch + 1].reshape(ch * W, K3), w3[1],
                            preferred_element_type=jnp.float32)
        acc = acc + jnp.dot(pw[2:ch + 2].reshape(ch * W, K3), w3[2],
                            preferred_element_type=jnp.float32)
        outs.append(jnp.maximum(acc + b, 0.0))
    return jnp.concatenate(outs, axis=0)


def _pad_hw(x, H, W, C):
    zr = jnp.zeros((1, W, C), x.dtype)
    zc = jnp.zeros((H + 2, 1, C), x.dtype)
    xp = jnp.concatenate([zr, x, zr], axis=0)
    return jnp.concatenate([zc, xp, zc], axis=1)


def _up_block_kernel(x1_ref, x2_ref, wh_ref, wwt_ref,
                     w1_ref, s1_ref, b1_ref, w2_ref, s2_ref, b2_ref, o_ref):
    _, H1, W1, C1 = x1_ref.shape
    _, H2, W2, C2 = x2_ref.shape
    Cmid = w1_ref.shape[-1]
    Cout = w2_ref.shape[-1]
    bf16 = jnp.bfloat16

    w1f = (w1_ref[...] * s1_ref[0][None, None, None, :]).astype(bf16)
    w1f = w1f.reshape(3, 3 * (C1 + C2), Cmid)
    w2f = (w2_ref[...] * s2_ref[0][None, None, None, :]).astype(bf16)
    w2f = w2f.reshape(3, 3 * Cmid, Cout)
    b1 = b1_ref[...]
    b2 = b2_ref[...]

    x1hwc = x1_ref[0].reshape(H1, W1 * C1)
    t = jnp.dot(wh_ref[...], x1hwc,
                preferred_element_type=jnp.float32)
    t = jnp.transpose(t.reshape(H2, W1, C1), (0, 2, 1))
    u = jnp.dot(t.reshape(H2 * C1, W1), wwt_ref[...],
                preferred_element_type=jnp.float32)
    up = jnp.transpose(u.reshape(H2, C1, W2), (0, 2, 1))

    xcat = jnp.concatenate([x2_ref[0].astype(bf16), up.astype(bf16)], axis=-1)
    xp = _pad_hw(xcat, H2, W2, C1 + C2)

    mid = _conv3x3_chunked(xp, w1f, b1, H2, W2)
    mp = _pad_hw(mid.astype(bf16).reshape(H2, W2, Cmid), H2, W2, Cmid)
    y = _conv3x3_chunked(mp, w2f, b2, H2, W2)

    o_ref[0] = y.reshape(H2, W2, Cout)


def kernel(x1_nchw, x2_nchw, w1, s1, b1, w2, s2, b2):
    N, C1, H1, W1 = x1_nchw.shape
    _, C2, H2, W2 = x2_nchw.shape
    Cin = C1 + C2
    Cmid = w1.shape[-1]
    Cout = w2.shape[-1]
    bf16 = jnp.bfloat16

    b1r = b1.reshape(1, Cmid).astype(jnp.float32)
    b2r = b2.reshape(1, Cout).astype(jnp.float32)
    s1r = s1.reshape(1, Cmid).astype(jnp.float32)
    s2r = s2.reshape(1, Cout).astype(jnp.float32)

    wh = _interp_mat(H2, H1)
    wwt = _interp_mat(W2, W1).T

    x1h = jnp.transpose(x1_nchw, (0, 2, 3, 1))
    x2h = jnp.transpose(x2_nchw, (0, 2, 3, 1))

    yh = pl.pallas_call(
        _up_block_kernel,
        out_shape=jax.ShapeDtypeStruct((N, H2, W2, Cout), x2_nchw.dtype),
        grid=(N,),
        in_specs=[
            pl.BlockSpec((1, H1, W1, C1), lambda n: (n, 0, 0, 0)),
            pl.BlockSpec((1, H2, W2, C2), lambda n: (n, 0, 0, 0)),
            pl.BlockSpec((H2, H1), lambda n: (0, 0)),
            pl.BlockSpec((W1, W2), lambda n: (0, 0)),
            pl.BlockSpec((3, 3, Cin, Cmid), lambda n: (0, 0, 0, 0)),
            pl.BlockSpec((1, Cmid), lambda n: (0, 0)),
            pl.BlockSpec((1, Cmid), lambda n: (0, 0)),
            pl.BlockSpec((3, 3, Cmid, Cout), lambda n: (0, 0, 0, 0)),
            pl.BlockSpec((1, Cout), lambda n: (0, 0)),
            pl.BlockSpec((1, Cout), lambda n: (0, 0)),
        ],
        out_specs=pl.BlockSpec((1, H2, W2, Cout), lambda n: (n, 0, 0, 0)),
        compiler_params=pltpu.CompilerParams(
            dimension_semantics=("parallel",),
            vmem_limit_bytes=_VMEM_LIMIT),
    )(x1h, x2h, wh, wwt, w1, s1r, b1r, w2, s2r, b2r)
    return jnp.transpose(yh, (0, 3, 1, 2))


# speedup vs baseline: 1.3034x; 1.1159x over previous
import jax
import jax.numpy as jnp
import numpy as np
from jax.experimental import pallas as pl
from jax.experimental.pallas import tpu as pltpu

_VMEM_LIMIT = 56 * 1024 * 1024


def _interp_mat(out_size, in_size):
    m = np.zeros((out_size, in_size), np.float32)
    for o in range(out_size):
        src = o * (in_size - 1) / (out_size - 1) if out_size > 1 else 0.0
        lo = int(np.floor(src))
        hi = min(lo + 1, in_size - 1)
        a = src - lo
        m[o, lo] += 1.0 - a
        m[o, hi] += a
    return jnp.asarray(m)


def _conv3x3_chunked(xp, w3, b, H, W):
    Cin = xp.shape[-1]
    K3 = 3 * Cin
    ch = 8 if H % 8 == 0 else H
    outs = []
    for r0 in range(0, H, ch):
        pw = jnp.concatenate(
            [xp[r0:r0 + ch + 2, kw:kw + W, :] for kw in range(3)], axis=-1)
        acc = jnp.dot(pw[0:ch].reshape(ch * W, K3), w3[0],
                      preferred_element_type=jnp.float32)
        acc = acc + jnp.dot(pw[1:ch + 1].reshape(ch * W, K3), w3[1],
                            preferred_element_type=jnp.float32)
        acc = acc + jnp.dot(pw[2:ch + 2].reshape(ch * W, K3), w3[2],
                            preferred_element_type=jnp.float32)
        outs.append(jnp.maximum(acc + b, 0.0))
    return jnp.concatenate(outs, axis=0)


def _pad_hw(x, H, W, C):
    zr = jnp.zeros((1, W, C), x.dtype)
    zc = jnp.zeros((H + 2, 1, C), x.dtype)
    xp = jnp.concatenate([zr, x, zr], axis=0)
    return jnp.concatenate([zc, xp, zc], axis=1)


def _up_block_kernel(x1_ref, x2_ref, wh_ref, wwt_ref,
                     w1_ref, s1_ref, b1_ref, w2_ref, s2_ref, b2_ref, o_ref):
    _, H1, W1, C1 = x1_ref.shape
    _, H2, W2, C2 = x2_ref.shape
    Cmid = w1_ref.shape[-1]
    Cout = w2_ref.shape[-1]
    bf16 = jnp.bfloat16

    w1f = (w1_ref[...] * s1_ref[0][None, None, None, :]).astype(bf16)
    w1f = w1f.reshape(3, 3 * (C1 + C2), Cmid)
    w2f = (w2_ref[...] * s2_ref[0][None, None, None, :]).astype(bf16)
    w2f = w2f.reshape(3, 3 * Cmid, Cout)
    b1 = b1_ref[...]
    b2 = b2_ref[...]

    x1hwc = x1_ref[0].reshape(H1, W1 * C1)
    t = jnp.dot(wh_ref[...], x1hwc,
                preferred_element_type=jnp.float32)
    t = jnp.transpose(t.reshape(H2, W1, C1), (0, 2, 1))
    u = jnp.dot(t.reshape(H2 * C1, W1), wwt_ref[...],
                preferred_element_type=jnp.float32)
    up = jnp.transpose(u.reshape(H2, C1, W2), (0, 2, 1))

    xcat = jnp.concatenate([x2_ref[0].astype(bf16), up.astype(bf16)], axis=-1)
    xp = _pad_hw(xcat, H2, W2, C1 + C2)

    mid = _conv3x3_chunked(xp, w1f, b1, H2, W2)
    mp = _pad_hw(mid.astype(bf16).reshape(H2, W2, Cmid), H2, W2, Cmid)
    y = _conv3x3_chunked(mp, w2f, b2, H2, W2)

    o_ref[0] = y.reshape(H2, W2, Cout)


def kernel(x1_nchw, x2_nchw, w1, s1, b1, w2, s2, b2):
    N, C1, H1, W1 = x1_nchw.shape
    _, C2, H2, W2 = x2_nchw.shape
    Cin = C1 + C2
    Cmid = w1.shape[-1]
    Cout = w2.shape[-1]

    b1r = b1.reshape(1, Cmid).astype(jnp.float32)
    b2r = b2.reshape(1, Cout).astype(jnp.float32)
    s1r = s1.reshape(1, Cmid).astype(jnp.float32)
    s2r = s2.reshape(1, Cout).astype(jnp.float32)

    wh = _interp_mat(H2, H1)
    wwt = _interp_mat(W2, W1).T

    x1h = jnp.transpose(x1_nchw, (0, 2, 3, 1))
    x2h = jnp.transpose(x2_nchw, (0, 2, 3, 1))

    yh = pl.pallas_call(
        _up_block_kernel,
        out_shape=jax.ShapeDtypeStruct((N, H2, W2, Cout), x2_nchw.dtype),
        grid=(N,),
        in_specs=[
            pl.BlockSpec((1, H1, W1, C1), lambda n: (n, 0, 0, 0)),
            pl.BlockSpec((1, H2, W2, C2), lambda n: (n, 0, 0, 0)),
            pl.BlockSpec((H2, H1), lambda n: (0, 0)),
            pl.BlockSpec((W1, W2), lambda n: (0, 0)),
            pl.BlockSpec((3, 3, Cin, Cmid), lambda n: (0, 0, 0, 0)),
            pl.BlockSpec((1, Cmid), lambda n: (0, 0)),
            pl.BlockSpec((1, Cmid), lambda n: (0, 0)),
            pl.BlockSpec((3, 3, Cmid, Cout), lambda n: (0, 0, 0, 0)),
            pl.BlockSpec((1, Cout), lambda n: (0, 0)),
            pl.BlockSpec((1, Cout), lambda n: (0, 0)),
        ],
        out_specs=pl.BlockSpec((1, H2, W2, Cout), lambda n: (n, 0, 0, 0)),
        compiler_params=pltpu.CompilerParams(
            dimension_semantics=("parallel",),
            vmem_limit_bytes=_VMEM_LIMIT),
    )(x1h, x2h, wh, wwt, w1, s1r, b1r, w2, s2r, b2r)
    return jnp.transpose(yh, (0, 3, 1, 2))
